# baseline (speedup 1.0000x reference)
import functools

import jax
import jax.numpy as jnp
from jax import lax
from jax.experimental import pallas as pl
from jax.experimental.pallas import tpu as pltpu
from jax.experimental.pallas import tpu_sc as plsc

SEQ, B = 4096, 4
NUM_SEGMENTS = 8
EMB_DIM = 128
OUT_DIM = 1024
N_TOKENS = SEQ * B

NC, NS = 2, 16
NW = NC * NS
TOK_PER_W = N_TOKENS // NW
CH = 32
NCH = TOK_PER_W // CH


def _p_kernel(table_ref, w_ref, p_ref):
    p_ref[...] = lax.dot_general(
        table_ref[...], w_ref[...],
        dimension_numbers=(((1,), (1,)), ((), ())),
        preferred_element_type=jnp.float32,
    )


def _sc_body(p_hbm, seg_hbm, out_hbm, idx_v, rows0, rows1, g0, g1, w0, w1):
    wid = lax.axis_index("s") * NC + lax.axis_index("c")
    base = wid * TOK_PER_W
    pltpu.sync_copy(seg_hbm.at[wid], idx_v)
    bufs = (rows0, rows1)
    gsems = (g0, g1)
    wsems = (w0, w1)
    gdesc = [None, None]
    wdesc = [None, None]
    gdesc[0] = pltpu.async_copy(p_hbm.at[idx_v.at[0]], rows0, g0)
    for j in range(NCH):
        b = j & 1
        nb = b ^ 1
        if j + 1 < NCH:
            if wdesc[nb] is not None:
                wdesc[nb].wait()
            gdesc[nb] = pltpu.async_copy(p_hbm.at[idx_v.at[j + 1]], bufs[nb], gsems[nb])
        gdesc[b].wait()
        wdesc[b] = pltpu.async_copy(bufs[b], out_hbm.at[pl.ds(base + j * CH, CH)], wsems[b])
    wdesc[0].wait()
    wdesc[1].wait()


@jax.jit
def kernel(input, align_pos, segment_ids, table, W):
    seg = segment_ids.astype(jnp.int32).reshape(NW, NCH, CH)
    P = pl.pallas_call(
        _p_kernel,
        out_shape=jax.ShapeDtypeStruct((NUM_SEGMENTS, OUT_DIM), jnp.float32),
    )(table, W)

    sc_gather = functools.partial(
        pl.kernel,
        out_type=jax.ShapeDtypeStruct((N_TOKENS, OUT_DIM), jnp.float32),
        mesh=plsc.VectorSubcoreMesh(core_axis_name="c", subcore_axis_name="s"),
        scratch_types=[
            pltpu.VMEM((NCH, CH), jnp.int32),
            pltpu.VMEM((CH, OUT_DIM), jnp.float32),
            pltpu.VMEM((CH, OUT_DIM), jnp.float32),
            pltpu.SemaphoreType.DMA,
            pltpu.SemaphoreType.DMA,
            pltpu.SemaphoreType.DMA,
            pltpu.SemaphoreType.DMA,
        ],
    )(_sc_body)
    out = sc_gather(P, seg)
    return out.reshape(SEQ, B, OUT_DIM)


# speedup vs baseline: 1.5319x; 1.5319x over previous
import functools

import jax
import jax.numpy as jnp
from jax import lax
from jax.experimental import pallas as pl
from jax.experimental.pallas import tpu as pltpu
from jax.experimental.pallas import tpu_sc as plsc

SEQ, B = 4096, 4
NUM_SEGMENTS = 8
EMB_DIM = 128
OUT_DIM = 1024
N_TOKENS = SEQ * B

NC, NS = 2, 16
NW = NC * NS
TOK_PER_W = N_TOKENS // NW
CH = 32
NCH = TOK_PER_W // CH
NBUF = 3


def _p_kernel(table_ref, w_ref, p_ref):
    p_ref[0] = lax.dot_general(
        table_ref[...], w_ref[...],
        dimension_numbers=(((1,), (1,)), ((), ())),
        preferred_element_type=jnp.float32,
    )


def _sc_body(p_hbm, seg_hbm, out_hbm, idx_v, rows0, rows1, rows2,
             g0, g1, g2, w0, w1, w2):
    wid = lax.axis_index("s") * NC + lax.axis_index("c")
    base = wid * TOK_PER_W
    pltpu.sync_copy(seg_hbm.at[wid], idx_v)
    bufs = (rows0, rows1, rows2)
    gsems = (g0, g1, g2)
    wsems = (w0, w1, w2)
    gdesc = [None] * NBUF
    wdesc = [None] * NBUF
    for j in range(NBUF - 1):
        gdesc[j] = pltpu.async_copy(p_hbm.at[idx_v.at[j]], bufs[j], gsems[j])
    for j in range(NCH):
        b = j % NBUF
        nb = (j + NBUF - 1) % NBUF
        if j + NBUF - 1 < NCH:
            if wdesc[nb] is not None:
                wdesc[nb].wait()
            gdesc[nb] = pltpu.async_copy(
                p_hbm.at[idx_v.at[j + NBUF - 1]], bufs[nb], gsems[nb])
        gdesc[b].wait()
        wdesc[b] = pltpu.async_copy(
            bufs[b], out_hbm.at[pl.ds(base + j * CH, CH)], wsems[b])
    for j in range(NBUF):
        if wdesc[j] is not None:
            wdesc[j].wait()


@jax.jit
def kernel(input, align_pos, segment_ids, table, W):
    seg = segment_ids.astype(jnp.int32).reshape(NW, NCH, CH)
    seg = seg + (jnp.arange(NW, dtype=jnp.int32) * NUM_SEGMENTS)[:, None, None]
    P = pl.pallas_call(
        _p_kernel,
        grid=(NW,),
        in_specs=[
            pl.BlockSpec((NUM_SEGMENTS, EMB_DIM), lambda i: (0, 0)),
            pl.BlockSpec((OUT_DIM, EMB_DIM), lambda i: (0, 0)),
        ],
        out_specs=pl.BlockSpec((1, NUM_SEGMENTS, OUT_DIM), lambda i: (i, 0, 0)),
        out_shape=jax.ShapeDtypeStruct((NW, NUM_SEGMENTS, OUT_DIM), jnp.float32),
    )(table, W)
    P = P.reshape(NW * NUM_SEGMENTS, OUT_DIM)

    sc_gather = functools.partial(
        pl.kernel,
        out_type=jax.ShapeDtypeStruct((N_TOKENS, OUT_DIM), jnp.float32),
        mesh=plsc.VectorSubcoreMesh(core_axis_name="c", subcore_axis_name="s"),
        scratch_types=[
            pltpu.VMEM((NCH, CH), jnp.int32),
            pltpu.VMEM((CH, OUT_DIM), jnp.float32),
            pltpu.VMEM((CH, OUT_DIM), jnp.float32),
            pltpu.VMEM((CH, OUT_DIM), jnp.float32),
            pltpu.SemaphoreType.DMA,
            pltpu.SemaphoreType.DMA,
            pltpu.SemaphoreType.DMA,
            pltpu.SemaphoreType.DMA,
            pltpu.SemaphoreType.DMA,
            pltpu.SemaphoreType.DMA,
        ],
    )(_sc_body)
    out = sc_gather(P, seg)
    return out.reshape(SEQ, B, OUT_DIM)
